# initial kernel scaffold (unmeasured)
import jax
import jax.numpy as jnp
from jax import lax
from jax.experimental import pallas as pl
from jax.experimental.pallas import tpu as pltpu

N_DEV = 4
M = 2048
K = 2048
F = 8192
FT = 512
N_FT = F // FT


def kernel(x, W1, W2):
    xb = x.astype(jnp.bfloat16)
    w1b = W1.astype(jnp.bfloat16)
    w2b = W2.astype(jnp.bfloat16)

    def body(x_ref, w1_ref, w2_ref, out_ref,
             xg, cbuf, rsbuf,
             xv, w1v, w2v, accv, rv,
             ldma, xsend, xrecv, asend, arecv):
        j = lax.axis_index("i")
        right = lax.rem(j + 1, N_DEV)
        left = lax.rem(j + 3, N_DEV)

        barrier_sem = pltpu.get_barrier_semaphore()
        for nbr in (left, right):
            pl.semaphore_signal(barrier_sem, inc=1, device_id=(nbr,),
                                device_id_type=pl.DeviceIdType.MESH)
        pl.semaphore_wait(barrier_sem, 2)

        for h in range(N_DEV - 1):
            b = lax.rem(j - h + N_DEV, N_DEV)
            src = x_ref if h == 0 else xg.at[b]
            rdma = pltpu.make_async_remote_copy(
                src_ref=src,
                dst_ref=xg.at[b],
                send_sem=xsend.at[h],
                recv_sem=xrecv.at[h],
                device_id=(right,),
                device_id_type=pl.DeviceIdType.MESH,
            )
            rdma.start()
            rdma.wait()

        for t in range(N_DEV):
            b = lax.rem(j - t + N_DEV, N_DEV)
            xsrc = x_ref if t == 0 else xg.at[b]
            cp = pltpu.make_async_copy(xsrc, xv, ldma)
            cp.start()
            cp.wait()
            accv[...] = jnp.zeros((M, K), jnp.float32)

            def ftile(ft, _):
                c1 = pltpu.make_async_copy(
                    w1_ref.at[:, pl.ds(ft * FT, FT)], w1v, ldma)
                c1.start()
                c1.wait()
                c2 = pltpu.make_async_copy(
                    w2_ref.at[pl.ds(ft * FT, FT), :], w2v, ldma)
                c2.start()
                c2.wait()
                h1 = jnp.dot(xv[...], w1v[...],
                             preferred_element_type=jnp.float32)
                h1 = h1 * (1.0 / (1.0 + jnp.exp(-h1)))
                accv[...] += jnp.dot(h1.astype(jnp.bfloat16), w2v[...],
                                     preferred_element_type=jnp.float32)
                return 0

            lax.fori_loop(0, N_FT, ftile, 0)
            cw = pltpu.make_async_copy(accv, cbuf.at[b], ldma)
            cw.start()
            cw.wait()

        for r in range(N_DEV - 1):
            b = lax.rem(j + r + 1, N_DEV)
            if r == 0:
                src = cbuf.at[b]
            else:
                ca = pltpu.make_async_copy(cbuf.at[b], accv, ldma)
                ca.start()
                ca.wait()
                cb = pltpu.make_async_copy(rsbuf.at[r - 1], rv, ldma)
                cb.start()
                cb.wait()
                accv[...] += rv[...]
                src = accv
            rdma = pltpu.make_async_remote_copy(
                src_ref=src,
                dst_ref=rsbuf.at[r],
                send_sem=asend.at[r],
                recv_sem=arecv.at[r],
                device_id=(left,),
                device_id_type=pl.DeviceIdType.MESH,
            )
            rdma.start()
            rdma.wait()

        ca = pltpu.make_async_copy(cbuf.at[j], accv, ldma)
        ca.start()
        ca.wait()
        cb = pltpu.make_async_copy(rsbuf.at[N_DEV - 2], rv, ldma)
        cb.start()
        cb.wait()
        accv[...] += rv[...]
        co = pltpu.make_async_copy(accv, out_ref, ldma)
        co.start()
        co.wait()

    return pl.pallas_call(
        body,
        out_shape=jax.ShapeDtypeStruct((M, K), jnp.float32),
        in_specs=[
            pl.BlockSpec(memory_space=pltpu.HBM),
            pl.BlockSpec(memory_space=pltpu.HBM),
            pl.BlockSpec(memory_space=pltpu.HBM),
        ],
        out_specs=pl.BlockSpec(memory_space=pltpu.HBM),
        scratch_shapes=[
            pltpu.HBM((N_DEV, M, K), jnp.bfloat16),
            pltpu.HBM((N_DEV, M, K), jnp.float32),
            pltpu.HBM((N_DEV - 1, M, K), jnp.float32),
            pltpu.VMEM((M, K), jnp.bfloat16),
            pltpu.VMEM((K, FT), jnp.bfloat16),
            pltpu.VMEM((FT, K), jnp.bfloat16),
            pltpu.VMEM((M, K), jnp.float32),
            pltpu.VMEM((M, K), jnp.float32),
            pltpu.SemaphoreType.DMA,
            pltpu.SemaphoreType.DMA((N_DEV - 1,)),
            pltpu.SemaphoreType.DMA((N_DEV - 1,)),
            pltpu.SemaphoreType.DMA((N_DEV - 1,)),
            pltpu.SemaphoreType.DMA((N_DEV - 1,)),
        ],
        compiler_params=pltpu.CompilerParams(collective_id=0),
    )(xb, w1b, w2b)


# baseline (device time: 1820395 ns/iter reference)
import jax
import jax.numpy as jnp
from jax import lax
from jax.experimental import pallas as pl
from jax.experimental.pallas import tpu as pltpu

N_DEV = 4
M = 2048
K = 2048
F = 8192
FT = 512
N_FT = F // FT


def kernel(x, W1, W2):
    xb = x.astype(jnp.bfloat16)
    w1b = W1.astype(jnp.bfloat16)
    w2b = W2.astype(jnp.bfloat16)

    def body(x_ref, w1_ref, w2_ref, out_ref,
             xg, cbuf, rsbuf,
             xv, w1v, w2v, accv, rv,
             ldma, xsend, xrecv, asend, arecv):
        j = lax.axis_index("i")
        right = lax.rem(j + 1, N_DEV)
        left = lax.rem(j + 3, N_DEV)

        barrier_sem = pltpu.get_barrier_semaphore()
        for nbr in (left, right):
            pl.semaphore_signal(barrier_sem, inc=1, device_id=(nbr,),
                                device_id_type=pl.DeviceIdType.MESH)
        pl.semaphore_wait(barrier_sem, 2)

        for h in range(N_DEV - 1):
            b = lax.rem(j - h + N_DEV, N_DEV)
            src = x_ref if h == 0 else xg.at[b]
            rdma = pltpu.make_async_remote_copy(
                src_ref=src,
                dst_ref=xg.at[b],
                send_sem=xsend.at[h],
                recv_sem=xrecv.at[h],
                device_id=(right,),
                device_id_type=pl.DeviceIdType.MESH,
            )
            rdma.start()
            rdma.wait()

        for t in range(N_DEV):
            b = lax.rem(j - t + N_DEV, N_DEV)
            xsrc = x_ref if t == 0 else xg.at[b]
            cp = pltpu.make_async_copy(xsrc, xv, ldma)
            cp.start()
            cp.wait()
            accv[...] = jnp.zeros((M, K), jnp.float32)

            def ftile(ft, _):
                c1 = pltpu.make_async_copy(
                    w1_ref.at[:, pl.ds(ft * FT, FT)], w1v, ldma)
                c1.start()
                c1.wait()
                c2 = pltpu.make_async_copy(
                    w2_ref.at[pl.ds(ft * FT, FT), :], w2v, ldma)
                c2.start()
                c2.wait()
                h1 = jnp.dot(xv[...], w1v[...],
                             preferred_element_type=jnp.float32)
                h1 = h1 * (1.0 / (1.0 + jnp.exp(-h1)))
                accv[...] += jnp.dot(h1.astype(jnp.bfloat16), w2v[...],
                                     preferred_element_type=jnp.float32)
                return 0

            lax.fori_loop(0, N_FT, ftile, 0)
            cw = pltpu.make_async_copy(accv, cbuf.at[b], ldma)
            cw.start()
            cw.wait()

        for r in range(N_DEV - 1):
            b = lax.rem(j + r + 1, N_DEV)
            if r == 0:
                src = cbuf.at[b]
            else:
                ca = pltpu.make_async_copy(cbuf.at[b], accv, ldma)
                ca.start()
                ca.wait()
                cb = pltpu.make_async_copy(rsbuf.at[r - 1], rv, ldma)
                cb.start()
                cb.wait()
                accv[...] += rv[...]
                src = accv
            rdma = pltpu.make_async_remote_copy(
                src_ref=src,
                dst_ref=rsbuf.at[r],
                send_sem=asend.at[r],
                recv_sem=arecv.at[r],
                device_id=(left,),
                device_id_type=pl.DeviceIdType.MESH,
            )
            rdma.start()
            rdma.wait()

        ca = pltpu.make_async_copy(cbuf.at[j], accv, ldma)
        ca.start()
        ca.wait()
        cb = pltpu.make_async_copy(rsbuf.at[N_DEV - 2], rv, ldma)
        cb.start()
        cb.wait()
        accv[...] += rv[...]
        co = pltpu.make_async_copy(accv, out_ref, ldma)
        co.start()
        co.wait()

    out, _, _, _ = pl.pallas_call(
        body,
        out_shape=[
            jax.ShapeDtypeStruct((M, K), jnp.float32),
            jax.ShapeDtypeStruct((N_DEV, M, K), jnp.bfloat16),
            jax.ShapeDtypeStruct((N_DEV, M, K), jnp.float32),
            jax.ShapeDtypeStruct((N_DEV - 1, M, K), jnp.float32),
        ],
        in_specs=[
            pl.BlockSpec(memory_space=pltpu.HBM),
            pl.BlockSpec(memory_space=pltpu.HBM),
            pl.BlockSpec(memory_space=pltpu.HBM),
        ],
        out_specs=[
            pl.BlockSpec(memory_space=pltpu.HBM),
            pl.BlockSpec(memory_space=pltpu.HBM),
            pl.BlockSpec(memory_space=pltpu.HBM),
            pl.BlockSpec(memory_space=pltpu.HBM),
        ],
        scratch_shapes=[
            pltpu.VMEM((M, K), jnp.bfloat16),
            pltpu.VMEM((K, FT), jnp.bfloat16),
            pltpu.VMEM((FT, K), jnp.bfloat16),
            pltpu.VMEM((M, K), jnp.float32),
            pltpu.VMEM((M, K), jnp.float32),
            pltpu.SemaphoreType.DMA,
            pltpu.SemaphoreType.DMA((N_DEV - 1,)),
            pltpu.SemaphoreType.DMA((N_DEV - 1,)),
            pltpu.SemaphoreType.DMA((N_DEV - 1,)),
            pltpu.SemaphoreType.DMA((N_DEV - 1,)),
        ],
        compiler_params=pltpu.CompilerParams(
            collective_id=0, vmem_limit_bytes=60 * 1024 * 1024),
    )(xb, w1b, w2b)
    return out


# device time: 985832 ns/iter; 1.8466x vs baseline; 1.8466x over previous
import jax
import jax.numpy as jnp
from jax import lax
from jax.experimental import pallas as pl
from jax.experimental.pallas import tpu as pltpu

N_DEV = 4
M = 2048
H = M // 2
K = 2048
F = 8192
FT = 512
N_FT = F // FT


def kernel(x, W1, W2):
    xb = x.astype(jnp.bfloat16)
    w1b = W1.astype(jnp.bfloat16)
    w2b = W2.astype(jnp.bfloat16)

    def body(x_ref, w1_ref, w2_ref,
             out_ref, xgA, xgB, aInA, aInB, aOutA, aOutB, cown,
             xcat, w1v, w2v, accv, rv,
             ldma, w1s, w2s,
             xsA, xrA, asA, arA, xsB, xrB, asB, arB):
        j = lax.axis_index("i")
        right = lax.rem(j + 1, N_DEV)
        left = lax.rem(j + 3, N_DEV)

        barrier_sem = pltpu.get_barrier_semaphore()
        for nbr in (left, right):
            pl.semaphore_signal(barrier_sem, inc=1, device_id=(nbr,),
                                device_id_type=pl.DeviceIdType.MESH)
        pl.semaphore_wait(barrier_sem, 2)

        send_descs = []

        def rdma(src, dst, ssem, rsem, dev):
            d = pltpu.make_async_remote_copy(
                src_ref=src, dst_ref=dst, send_sem=ssem, recv_sem=rsem,
                device_id=(dev,), device_id_type=pl.DeviceIdType.MESH)
            d.start()
            send_descs.append(d)
            return d

        def copy(src, dst):
            c = pltpu.make_async_copy(src, dst, ldma)
            c.start()
            c.wait()

        top = pl.ds(0, H)
        bot = pl.ds(H, H)

        def compute_step():
            pltpu.make_async_copy(
                w1_ref.at[:, pl.ds(0, FT)], w1v.at[0], w1s.at[0]).start()
            pltpu.make_async_copy(
                w2_ref.at[pl.ds(0, FT), :], w2v.at[0], w2s.at[0]).start()

            def ftile(ft, _):
                slot = lax.rem(ft, 2)
                nslot = 1 - slot

                @pl.when(ft + 1 < N_FT)
                def _():
                    pltpu.make_async_copy(
                        w1_ref.at[:, pl.ds((ft + 1) * FT, FT)],
                        w1v.at[nslot], w1s.at[nslot]).start()
                    pltpu.make_async_copy(
                        w2_ref.at[pl.ds((ft + 1) * FT, FT), :],
                        w2v.at[nslot], w2s.at[nslot]).start()

                pltpu.make_async_copy(
                    w1_ref.at[:, pl.ds(ft * FT, FT)],
                    w1v.at[slot], w1s.at[slot]).wait()
                pltpu.make_async_copy(
                    w2_ref.at[pl.ds(ft * FT, FT), :],
                    w2v.at[slot], w2s.at[slot]).wait()

                h1 = jnp.dot(xcat[...], w1v[slot],
                             preferred_element_type=jnp.float32)
                h1 = h1 * (1.0 / (1.0 + jnp.exp(-h1)))
                contrib = jnp.dot(h1.astype(jnp.bfloat16), w2v[slot],
                                  preferred_element_type=jnp.float32)

                @pl.when(ft == 0)
                def _():
                    accv[...] = contrib

                @pl.when(ft != 0)
                def _():
                    accv[...] += contrib

                return 0

            lax.fori_loop(0, N_FT, ftile, 0)

        dxA = [rdma(x_ref.at[top], xgA.at[0], xsA.at[0], xrA.at[0], right)]
        dxB = [rdma(x_ref.at[bot], xgB.at[0], xsB.at[0], xrB.at[0], left)]
        copy(x_ref, xcat)
        compute_step()
        copy(accv.at[top], cown.at[0])
        copy(accv.at[bot], cown.at[1])

        daA, daB = [], []
        for t in (1, 2, 3):
            dxA[t - 1].wait_recv()
            dxB[t - 1].wait_recv()
            if t < 3:
                dxA.append(rdma(xgA.at[t - 1], xgA.at[t],
                                xsA.at[t], xrA.at[t], right))
                dxB.append(rdma(xgB.at[t - 1], xgB.at[t],
                                xsB.at[t], xrB.at[t], left))
            copy(xgA.at[t - 1], xcat.at[top])
            copy(xgB.at[t - 1], xcat.at[bot])
            compute_step()
            if t >= 2:
                daA[t - 2].wait_recv()
                copy(aInA.at[t - 2], rv)
                accv[top] += rv[...]
                daB[t - 2].wait_recv()
                copy(aInB.at[t - 2], rv)
                accv[bot] += rv[...]
            copy(accv.at[top], aOutA.at[t - 1])
            daA.append(rdma(aOutA.at[t - 1], aInA.at[t - 1],
                            asA.at[t - 1], arA.at[t - 1], right))
            copy(accv.at[bot], aOutB.at[t - 1])
            daB.append(rdma(aOutB.at[t - 1], aInB.at[t - 1],
                            asB.at[t - 1], arB.at[t - 1], left))

        daA[2].wait_recv()
        copy(aInA.at[2], rv)
        copy(cown.at[0], accv.at[top])
        accv[top] += rv[...]
        copy(accv.at[top], out_ref.at[top])

        daB[2].wait_recv()
        copy(aInB.at[2], rv)
        copy(cown.at[1], accv.at[bot])
        accv[bot] += rv[...]
        copy(accv.at[bot], out_ref.at[bot])

        for d in send_descs:
            d.wait_send()

    out, *_ = pl.pallas_call(
        body,
        out_shape=[
            jax.ShapeDtypeStruct((M, K), jnp.float32),
            jax.ShapeDtypeStruct((N_DEV - 1, H, K), jnp.bfloat16),
            jax.ShapeDtypeStruct((N_DEV - 1, H, K), jnp.bfloat16),
            jax.ShapeDtypeStruct((N_DEV - 1, H, K), jnp.float32),
            jax.ShapeDtypeStruct((N_DEV - 1, H, K), jnp.float32),
            jax.ShapeDtypeStruct((N_DEV - 1, H, K), jnp.float32),
            jax.ShapeDtypeStruct((N_DEV - 1, H, K), jnp.float32),
            jax.ShapeDtypeStruct((2, H, K), jnp.float32),
        ],
        in_specs=[
            pl.BlockSpec(memory_space=pltpu.HBM),
            pl.BlockSpec(memory_space=pltpu.HBM),
            pl.BlockSpec(memory_space=pltpu.HBM),
        ],
        out_specs=[pl.BlockSpec(memory_space=pltpu.HBM)] * 8,
        scratch_shapes=[
            pltpu.VMEM((M, K), jnp.bfloat16),
            pltpu.VMEM((2, K, FT), jnp.bfloat16),
            pltpu.VMEM((2, FT, K), jnp.bfloat16),
            pltpu.VMEM((M, K), jnp.float32),
            pltpu.VMEM((H, K), jnp.float32),
            pltpu.SemaphoreType.DMA,
            pltpu.SemaphoreType.DMA((2,)),
            pltpu.SemaphoreType.DMA((2,)),
            pltpu.SemaphoreType.DMA((N_DEV - 1,)),
            pltpu.SemaphoreType.DMA((N_DEV - 1,)),
            pltpu.SemaphoreType.DMA((N_DEV - 1,)),
            pltpu.SemaphoreType.DMA((N_DEV - 1,)),
            pltpu.SemaphoreType.DMA((N_DEV - 1,)),
            pltpu.SemaphoreType.DMA((N_DEV - 1,)),
            pltpu.SemaphoreType.DMA((N_DEV - 1,)),
            pltpu.SemaphoreType.DMA((N_DEV - 1,)),
        ],
        compiler_params=pltpu.CompilerParams(
            collective_id=0, vmem_limit_bytes=60 * 1024 * 1024),
    )(xb, w1b, w2b)
    return out


# device time: 922226 ns/iter; 1.9739x vs baseline; 1.0690x over previous
import jax
import jax.numpy as jnp
from jax import lax
from jax.experimental import pallas as pl
from jax.experimental.pallas import tpu as pltpu

N_DEV = 4
M = 2048
H = M // 2
K = 2048
F = 8192
FT = 512
N_FT = F // FT


def kernel(x, W1, W2):
    xb = x.astype(jnp.bfloat16)
    w1b = W1.astype(jnp.bfloat16)
    w2b = W2.astype(jnp.bfloat16)

    def body(x_ref, w1_ref, w2_ref,
             out_ref, xgA, xgB, aInA, aInB, aOutA, aOutB, cown,
             xcat, w1v, w2v, accv, rv, sv,
             ldma, w1s, w2s,
             xsA, xrA, asA, arA, xsB, xrB, asB, arB):
        j = lax.axis_index("i")
        right = lax.rem(j + 1, N_DEV)
        left = lax.rem(j + 3, N_DEV)

        barrier_sem = pltpu.get_barrier_semaphore()
        for nbr in (left, right):
            pl.semaphore_signal(barrier_sem, inc=1, device_id=(nbr,),
                                device_id_type=pl.DeviceIdType.MESH)
        pl.semaphore_wait(barrier_sem, 2)

        send_descs = []

        def rdma(src, dst, ssem, rsem, dev):
            d = pltpu.make_async_remote_copy(
                src_ref=src, dst_ref=dst, send_sem=ssem, recv_sem=rsem,
                device_id=(dev,), device_id_type=pl.DeviceIdType.MESH)
            d.start()
            send_descs.append(d)
            return d

        def copy(src, dst):
            c = pltpu.make_async_copy(src, dst, ldma)
            c.start()
            c.wait()

        top = pl.ds(0, H)
        bot = pl.ds(H, H)

        def compute_step():
            pltpu.make_async_copy(
                w1_ref.at[:, pl.ds(0, FT)], w1v.at[0], w1s.at[0]).start()
            pltpu.make_async_copy(
                w2_ref.at[pl.ds(0, FT), :], w2v.at[0], w2s.at[0]).start()

            def ftile(ft, _):
                slot = lax.rem(ft, 2)
                nslot = 1 - slot

                @pl.when(ft + 1 < N_FT)
                def _():
                    pltpu.make_async_copy(
                        w1_ref.at[:, pl.ds((ft + 1) * FT, FT)],
                        w1v.at[nslot], w1s.at[nslot]).start()
                    pltpu.make_async_copy(
                        w2_ref.at[pl.ds((ft + 1) * FT, FT), :],
                        w2v.at[nslot], w2s.at[nslot]).start()

                pltpu.make_async_copy(
                    w1_ref.at[:, pl.ds(ft * FT, FT)],
                    w1v.at[slot], w1s.at[slot]).wait()
                pltpu.make_async_copy(
                    w2_ref.at[pl.ds(ft * FT, FT), :],
                    w2v.at[slot], w2s.at[slot]).wait()

                h1 = jnp.dot(xcat[...], w1v[slot],
                             preferred_element_type=jnp.float32)
                h1 = h1 * 0.5 * (1.0 + jnp.tanh(h1 * 0.5))
                contrib = jnp.dot(h1.astype(jnp.bfloat16), w2v[slot],
                                  preferred_element_type=jnp.float32)

                @pl.when(ft == 0)
                def _():
                    accv[...] = contrib

                @pl.when(ft != 0)
                def _():
                    accv[...] += contrib

                return 0

            lax.fori_loop(0, N_FT, ftile, 0)

        dxA = [rdma(x_ref.at[top], xgA.at[0], xsA.at[0], xrA.at[0], right)]
        dxB = [rdma(x_ref.at[bot], xgB.at[0], xsB.at[0], xrB.at[0], left)]
        copy(x_ref, xcat)
        compute_step()
        copy(accv.at[top], cown.at[0])
        copy(accv.at[bot], cown.at[1])

        daA, daB = [], []
        for t in (1, 2, 3):
            dxA[t - 1].wait_recv()
            dxB[t - 1].wait_recv()
            if t < 3:
                dxA.append(rdma(xgA.at[t - 1], xgA.at[t],
                                xsA.at[t], xrA.at[t], right))
                dxB.append(rdma(xgB.at[t - 1], xgB.at[t],
                                xsB.at[t], xrB.at[t], left))
            copy(xgA.at[t - 1], xcat.at[top])
            copy(xgB.at[t - 1], xcat.at[bot])
            compute_step()
            if t >= 2:
                daA[t - 2].wait_recv()
                copy(aInA.at[t - 2], rv)
                accv[top] += rv[...].astype(jnp.float32)
                daB[t - 2].wait_recv()
                copy(aInB.at[t - 2], rv)
                accv[bot] += rv[...].astype(jnp.float32)
            sv[...] = accv[top].astype(jnp.bfloat16)
            copy(sv, aOutA.at[t - 1])
            daA.append(rdma(aOutA.at[t - 1], aInA.at[t - 1],
                            asA.at[t - 1], arA.at[t - 1], right))
            sv[...] = accv[bot].astype(jnp.bfloat16)
            copy(sv, aOutB.at[t - 1])
            daB.append(rdma(aOutB.at[t - 1], aInB.at[t - 1],
                            asB.at[t - 1], arB.at[t - 1], left))

        daA[2].wait_recv()
        copy(aInA.at[2], rv)
        copy(cown.at[0], accv.at[top])
        accv[top] += rv[...].astype(jnp.float32)
        copy(accv.at[top], out_ref.at[top])

        daB[2].wait_recv()
        copy(aInB.at[2], rv)
        copy(cown.at[1], accv.at[bot])
        accv[bot] += rv[...].astype(jnp.float32)
        copy(accv.at[bot], out_ref.at[bot])

        for d in send_descs:
            d.wait_send()

    out, *_ = pl.pallas_call(
        body,
        out_shape=[
            jax.ShapeDtypeStruct((M, K), jnp.float32),
            jax.ShapeDtypeStruct((N_DEV - 1, H, K), jnp.bfloat16),
            jax.ShapeDtypeStruct((N_DEV - 1, H, K), jnp.bfloat16),
            jax.ShapeDtypeStruct((N_DEV - 1, H, K), jnp.bfloat16),
            jax.ShapeDtypeStruct((N_DEV - 1, H, K), jnp.bfloat16),
            jax.ShapeDtypeStruct((N_DEV - 1, H, K), jnp.bfloat16),
            jax.ShapeDtypeStruct((N_DEV - 1, H, K), jnp.bfloat16),
            jax.ShapeDtypeStruct((2, H, K), jnp.float32),
        ],
        in_specs=[
            pl.BlockSpec(memory_space=pltpu.HBM),
            pl.BlockSpec(memory_space=pltpu.HBM),
            pl.BlockSpec(memory_space=pltpu.HBM),
        ],
        out_specs=[pl.BlockSpec(memory_space=pltpu.HBM)] * 8,
        scratch_shapes=[
            pltpu.VMEM((M, K), jnp.bfloat16),
            pltpu.VMEM((2, K, FT), jnp.bfloat16),
            pltpu.VMEM((2, FT, K), jnp.bfloat16),
            pltpu.VMEM((M, K), jnp.float32),
            pltpu.VMEM((H, K), jnp.bfloat16),
            pltpu.VMEM((H, K), jnp.bfloat16),
            pltpu.SemaphoreType.DMA,
            pltpu.SemaphoreType.DMA((2,)),
            pltpu.SemaphoreType.DMA((2,)),
            pltpu.SemaphoreType.DMA((N_DEV - 1,)),
            pltpu.SemaphoreType.DMA((N_DEV - 1,)),
            pltpu.SemaphoreType.DMA((N_DEV - 1,)),
            pltpu.SemaphoreType.DMA((N_DEV - 1,)),
            pltpu.SemaphoreType.DMA((N_DEV - 1,)),
            pltpu.SemaphoreType.DMA((N_DEV - 1,)),
            pltpu.SemaphoreType.DMA((N_DEV - 1,)),
            pltpu.SemaphoreType.DMA((N_DEV - 1,)),
        ],
        compiler_params=pltpu.CompilerParams(
            collective_id=0, vmem_limit_bytes=60 * 1024 * 1024),
    )(xb, w1b, w2b)
    return out


# device time: 850777 ns/iter; 2.1397x vs baseline; 1.0840x over previous
import jax
import jax.numpy as jnp
from jax import lax
from jax.experimental import pallas as pl
from jax.experimental.pallas import tpu as pltpu

N_DEV = 4
M = 2048
H = M // 2
K = 2048
F = 8192
FT = 1024
N_FT = F // FT


def kernel(x, W1, W2):
    xb = x.astype(jnp.bfloat16)
    w1b = W1.astype(jnp.bfloat16)
    w2b = W2.astype(jnp.bfloat16)

    def body(x_ref, w1_ref, w2_ref,
             out_ref, xgA, xgB, aInA, aInB, aOutA, aOutB, cown,
             xcat, w1v, w2v, accv, rv, sv,
             ldma, w1s, w2s,
             xsA, xrA, asA, arA, xsB, xrB, asB, arB):
        j = lax.axis_index("i")
        right = lax.rem(j + 1, N_DEV)
        left = lax.rem(j + 3, N_DEV)

        barrier_sem = pltpu.get_barrier_semaphore()
        for nbr in (left, right):
            pl.semaphore_signal(barrier_sem, inc=1, device_id=(nbr,),
                                device_id_type=pl.DeviceIdType.MESH)
        pl.semaphore_wait(barrier_sem, 2)

        send_descs = []

        def rdma(src, dst, ssem, rsem, dev):
            d = pltpu.make_async_remote_copy(
                src_ref=src, dst_ref=dst, send_sem=ssem, recv_sem=rsem,
                device_id=(dev,), device_id_type=pl.DeviceIdType.MESH)
            d.start()
            send_descs.append(d)
            return d

        def copy(src, dst):
            c = pltpu.make_async_copy(src, dst, ldma)
            c.start()
            c.wait()

        top = pl.ds(0, H)
        bot = pl.ds(H, H)

        def compute_step():
            pltpu.make_async_copy(
                w1_ref.at[:, pl.ds(0, FT)], w1v.at[0], w1s.at[0]).start()
            pltpu.make_async_copy(
                w2_ref.at[pl.ds(0, FT), :], w2v.at[0], w2s.at[0]).start()

            def ftile(ft, _):
                slot = lax.rem(ft, 2)
                nslot = 1 - slot

                @pl.when(ft + 1 < N_FT)
                def _():
                    pltpu.make_async_copy(
                        w1_ref.at[:, pl.ds((ft + 1) * FT, FT)],
                        w1v.at[nslot], w1s.at[nslot]).start()
                    pltpu.make_async_copy(
                        w2_ref.at[pl.ds((ft + 1) * FT, FT), :],
                        w2v.at[nslot], w2s.at[nslot]).start()

                pltpu.make_async_copy(
                    w1_ref.at[:, pl.ds(ft * FT, FT)],
                    w1v.at[slot], w1s.at[slot]).wait()
                pltpu.make_async_copy(
                    w2_ref.at[pl.ds(ft * FT, FT), :],
                    w2v.at[slot], w2s.at[slot]).wait()

                for rows in (top, bot):
                    h1 = jnp.dot(xcat[rows], w1v[slot],
                                 preferred_element_type=jnp.float32)
                    h1 = h1 * 0.5 * (1.0 + jnp.tanh(h1 * 0.5))
                    contrib = jnp.dot(h1.astype(jnp.bfloat16), w2v[slot],
                                      preferred_element_type=jnp.float32)

                    @pl.when(ft == 0)
                    def _():
                        accv[rows] = contrib

                    @pl.when(ft != 0)
                    def _():
                        accv[rows] += contrib

                return 0

            lax.fori_loop(0, N_FT, ftile, 0)

        dxA = [rdma(x_ref.at[top], xgA.at[0], xsA.at[0], xrA.at[0], right)]
        dxB = [rdma(x_ref.at[bot], xgB.at[0], xsB.at[0], xrB.at[0], left)]
        copy(x_ref, xcat)
        compute_step()
        copy(accv.at[top], cown.at[0])
        copy(accv.at[bot], cown.at[1])

        daA, daB = [], []
        for t in (1, 2, 3):
            dxA[t - 1].wait_recv()
            dxB[t - 1].wait_recv()
            if t < 3:
                dxA.append(rdma(xgA.at[t - 1], xgA.at[t],
                                xsA.at[t], xrA.at[t], right))
                dxB.append(rdma(xgB.at[t - 1], xgB.at[t],
                                xsB.at[t], xrB.at[t], left))
            copy(xgA.at[t - 1], xcat.at[top])
            copy(xgB.at[t - 1], xcat.at[bot])
            compute_step()
            if t >= 2:
                daA[t - 2].wait_recv()
                copy(aInA.at[t - 2], rv)
                accv[top] += rv[...].astype(jnp.float32)
                daB[t - 2].wait_recv()
                copy(aInB.at[t - 2], rv)
                accv[bot] += rv[...].astype(jnp.float32)
            sv[...] = accv[top].astype(jnp.bfloat16)
            copy(sv, aOutA.at[t - 1])
            daA.append(rdma(aOutA.at[t - 1], aInA.at[t - 1],
                            asA.at[t - 1], arA.at[t - 1], right))
            sv[...] = accv[bot].astype(jnp.bfloat16)
            copy(sv, aOutB.at[t - 1])
            daB.append(rdma(aOutB.at[t - 1], aInB.at[t - 1],
                            asB.at[t - 1], arB.at[t - 1], left))

        daA[2].wait_recv()
        copy(aInA.at[2], rv)
        copy(cown.at[0], accv.at[top])
        accv[top] += rv[...].astype(jnp.float32)
        copy(accv.at[top], out_ref.at[top])

        daB[2].wait_recv()
        copy(aInB.at[2], rv)
        copy(cown.at[1], accv.at[bot])
        accv[bot] += rv[...].astype(jnp.float32)
        copy(accv.at[bot], out_ref.at[bot])

        for d in send_descs:
            d.wait_send()

    out, *_ = pl.pallas_call(
        body,
        out_shape=[
            jax.ShapeDtypeStruct((M, K), jnp.float32),
            jax.ShapeDtypeStruct((N_DEV - 1, H, K), jnp.bfloat16),
            jax.ShapeDtypeStruct((N_DEV - 1, H, K), jnp.bfloat16),
            jax.ShapeDtypeStruct((N_DEV - 1, H, K), jnp.bfloat16),
            jax.ShapeDtypeStruct((N_DEV - 1, H, K), jnp.bfloat16),
            jax.ShapeDtypeStruct((N_DEV - 1, H, K), jnp.bfloat16),
            jax.ShapeDtypeStruct((N_DEV - 1, H, K), jnp.bfloat16),
            jax.ShapeDtypeStruct((2, H, K), jnp.float32),
        ],
        in_specs=[
            pl.BlockSpec(memory_space=pltpu.HBM),
            pl.BlockSpec(memory_space=pltpu.HBM),
            pl.BlockSpec(memory_space=pltpu.HBM),
        ],
        out_specs=[pl.BlockSpec(memory_space=pltpu.HBM)] * 8,
        scratch_shapes=[
            pltpu.VMEM((M, K), jnp.bfloat16),
            pltpu.VMEM((2, K, FT), jnp.bfloat16),
            pltpu.VMEM((2, FT, K), jnp.bfloat16),
            pltpu.VMEM((M, K), jnp.float32),
            pltpu.VMEM((H, K), jnp.bfloat16),
            pltpu.VMEM((H, K), jnp.bfloat16),
            pltpu.SemaphoreType.DMA,
            pltpu.SemaphoreType.DMA((2,)),
            pltpu.SemaphoreType.DMA((2,)),
            pltpu.SemaphoreType.DMA((N_DEV - 1,)),
            pltpu.SemaphoreType.DMA((N_DEV - 1,)),
            pltpu.SemaphoreType.DMA((N_DEV - 1,)),
            pltpu.SemaphoreType.DMA((N_DEV - 1,)),
            pltpu.SemaphoreType.DMA((N_DEV - 1,)),
            pltpu.SemaphoreType.DMA((N_DEV - 1,)),
            pltpu.SemaphoreType.DMA((N_DEV - 1,)),
            pltpu.SemaphoreType.DMA((N_DEV - 1,)),
        ],
        compiler_params=pltpu.CompilerParams(
            collective_id=0, vmem_limit_bytes=63 * 1024 * 1024),
    )(xb, w1b, w2b)
    return out


# device time: 827993 ns/iter; 2.1986x vs baseline; 1.0275x over previous
import jax
import jax.numpy as jnp
from jax import lax
from jax.experimental import pallas as pl
from jax.experimental.pallas import tpu as pltpu

N_DEV = 4
M = 2048
H = M // 2
K = 2048
F = 8192
FT = 1024
N_FT = F // FT


def kernel(x, W1, W2):
    xb = x.astype(jnp.bfloat16)
    w1b = W1.astype(jnp.bfloat16)
    w2b = W2.astype(jnp.bfloat16)

    def body(x_ref, w1_ref, w2_ref,
             out_ref, xgA, xgB, aInA, aInB, cown,
             xcat, w1v, w2v, accv, mixA, mixB,
             cs, w1s, w2s,
             xsA, xrA, asA, arA, xsB, xrB, asB, arB):
        j = lax.axis_index("i")
        right = lax.rem(j + 1, N_DEV)
        left = lax.rem(j + 3, N_DEV)

        barrier_sem = pltpu.get_barrier_semaphore()
        for nbr in (left, right):
            pl.semaphore_signal(barrier_sem, inc=1, device_id=(nbr,),
                                device_id_type=pl.DeviceIdType.MESH)
        pl.semaphore_wait(barrier_sem, 2)

        x_descs = []

        def rdma(src, dst, ssem, rsem, dev, track=True):
            d = pltpu.make_async_remote_copy(
                src_ref=src, dst_ref=dst, send_sem=ssem, recv_sem=rsem,
                device_id=(dev,), device_id_type=pl.DeviceIdType.MESH)
            d.start()
            if track:
                x_descs.append(d)
            return d

        def copy_start(src, dst, sem):
            c = pltpu.make_async_copy(src, dst, sem)
            c.start()
            return c

        top = pl.ds(0, H)
        bot = pl.ds(H, H)

        def compute_step():
            pltpu.make_async_copy(
                w1_ref.at[:, pl.ds(0, FT)], w1v.at[0], w1s.at[0]).start()
            pltpu.make_async_copy(
                w2_ref.at[pl.ds(0, FT), :], w2v.at[0], w2s.at[0]).start()

            def ftile(ft, _):
                slot = lax.rem(ft, 2)
                nslot = 1 - slot

                @pl.when(ft + 1 < N_FT)
                def _():
                    pltpu.make_async_copy(
                        w1_ref.at[:, pl.ds((ft + 1) * FT, FT)],
                        w1v.at[nslot], w1s.at[nslot]).start()
                    pltpu.make_async_copy(
                        w2_ref.at[pl.ds((ft + 1) * FT, FT), :],
                        w2v.at[nslot], w2s.at[nslot]).start()

                pltpu.make_async_copy(
                    w1_ref.at[:, pl.ds(ft * FT, FT)],
                    w1v.at[slot], w1s.at[slot]).wait()
                pltpu.make_async_copy(
                    w2_ref.at[pl.ds(ft * FT, FT), :],
                    w2v.at[slot], w2s.at[slot]).wait()

                for rows in (top, bot):
                    h1 = jnp.dot(xcat[rows], w1v[slot],
                                 preferred_element_type=jnp.float32)
                    h1 = h1 * 0.5 * (1.0 + jnp.tanh(h1 * 0.5))
                    contrib = jnp.dot(h1.astype(jnp.bfloat16), w2v[slot],
                                      preferred_element_type=jnp.float32)

                    @pl.when(ft == 0)
                    def _():
                        accv[rows] = contrib

                    @pl.when(ft != 0)
                    def _():
                        accv[rows] += contrib

                return 0

            lax.fori_loop(0, N_FT, ftile, 0)

        dxA = [rdma(x_ref.at[top], xgA.at[0], xsA.at[0], xrA.at[0], right)]
        dxB = [rdma(x_ref.at[bot], xgB.at[0], xsB.at[0], xrB.at[0], left)]
        copy_start(x_ref, xcat, cs.at[0]).wait()
        compute_step()
        c0 = copy_start(accv.at[top], cown.at[0], cs.at[0])
        c1 = copy_start(accv.at[bot], cown.at[1], cs.at[1])
        c0.wait()
        c1.wait()

        daA, daB = [], []
        for t in (1, 2, 3):
            dxA[t - 1].wait_recv()
            dxB[t - 1].wait_recv()
            if t < 3:
                dxA.append(rdma(xgA.at[t - 1], xgA.at[t],
                                xsA.at[t], xrA.at[t], right))
                dxB.append(rdma(xgB.at[t - 1], xgB.at[t],
                                xsB.at[t], xrB.at[t], left))
            c0 = copy_start(xgA.at[t - 1], xcat.at[top], cs.at[0])
            c1 = copy_start(xgB.at[t - 1], xcat.at[bot], cs.at[1])
            c0.wait()
            c1.wait()
            compute_step()
            if t >= 2:
                daA[t - 2].wait_send()
                daB[t - 2].wait_send()
                daA[t - 2].wait_recv()
                daB[t - 2].wait_recv()
                c0 = copy_start(aInA.at[t - 2], mixA, cs.at[0])
                c1 = copy_start(aInB.at[t - 2], mixB, cs.at[1])
                c0.wait()
                accv[top] += mixA[...].astype(jnp.float32)
                c1.wait()
                accv[bot] += mixB[...].astype(jnp.float32)
            mixA[...] = accv[top].astype(jnp.bfloat16)
            daA.append(rdma(mixA, aInA.at[t - 1],
                            asA.at[t - 1], arA.at[t - 1], right, track=False))
            mixB[...] = accv[bot].astype(jnp.bfloat16)
            daB.append(rdma(mixB, aInB.at[t - 1],
                            asB.at[t - 1], arB.at[t - 1], left, track=False))

        daA[2].wait_send()
        daB[2].wait_send()
        daA[2].wait_recv()
        daB[2].wait_recv()
        c0 = copy_start(aInA.at[2], mixA, cs.at[0])
        c1 = copy_start(aInB.at[2], mixB, cs.at[1])
        c2 = copy_start(cown.at[0], accv.at[top], cs.at[2])
        c3 = copy_start(cown.at[1], accv.at[bot], cs.at[3])
        c0.wait()
        c2.wait()
        accv[top] += mixA[...].astype(jnp.float32)
        c1.wait()
        c3.wait()
        accv[bot] += mixB[...].astype(jnp.float32)
        c0 = copy_start(accv.at[top], out_ref.at[top], cs.at[0])
        c1 = copy_start(accv.at[bot], out_ref.at[bot], cs.at[1])
        c0.wait()
        c1.wait()

        for d in x_descs:
            d.wait_send()

    out, *_ = pl.pallas_call(
        body,
        out_shape=[
            jax.ShapeDtypeStruct((M, K), jnp.float32),
            jax.ShapeDtypeStruct((N_DEV - 1, H, K), jnp.bfloat16),
            jax.ShapeDtypeStruct((N_DEV - 1, H, K), jnp.bfloat16),
            jax.ShapeDtypeStruct((N_DEV - 1, H, K), jnp.bfloat16),
            jax.ShapeDtypeStruct((N_DEV - 1, H, K), jnp.bfloat16),
            jax.ShapeDtypeStruct((2, H, K), jnp.float32),
        ],
        in_specs=[
            pl.BlockSpec(memory_space=pltpu.HBM),
            pl.BlockSpec(memory_space=pltpu.HBM),
            pl.BlockSpec(memory_space=pltpu.HBM),
        ],
        out_specs=[pl.BlockSpec(memory_space=pltpu.HBM)] * 6,
        scratch_shapes=[
            pltpu.VMEM((M, K), jnp.bfloat16),
            pltpu.VMEM((2, K, FT), jnp.bfloat16),
            pltpu.VMEM((2, FT, K), jnp.bfloat16),
            pltpu.VMEM((M, K), jnp.float32),
            pltpu.VMEM((H, K), jnp.bfloat16),
            pltpu.VMEM((H, K), jnp.bfloat16),
            pltpu.SemaphoreType.DMA((4,)),
            pltpu.SemaphoreType.DMA((2,)),
            pltpu.SemaphoreType.DMA((2,)),
            pltpu.SemaphoreType.DMA((N_DEV - 1,)),
            pltpu.SemaphoreType.DMA((N_DEV - 1,)),
            pltpu.SemaphoreType.DMA((N_DEV - 1,)),
            pltpu.SemaphoreType.DMA((N_DEV - 1,)),
            pltpu.SemaphoreType.DMA((N_DEV - 1,)),
            pltpu.SemaphoreType.DMA((N_DEV - 1,)),
            pltpu.SemaphoreType.DMA((N_DEV - 1,)),
            pltpu.SemaphoreType.DMA((N_DEV - 1,)),
        ],
        compiler_params=pltpu.CompilerParams(
            collective_id=0, vmem_limit_bytes=63 * 1024 * 1024),
    )(xb, w1b, w2b)
    return out
